# baseline (speedup 1.0000x reference)
import functools

import jax
import jax.numpy as jnp
from jax import lax
from jax.experimental import pallas as pl
from jax.experimental.pallas import tpu as pltpu
from jax.experimental.pallas import tpu_sc as plsc

N_IN = 16384
N_OUT = 16384
BATCH = 1024
DEGREE = 2

NC, NS, L = 2, 16, 16
NW = NC * NS
ROWS_PER_W = BATCH // NW
PAIRS_PER_W = ROWS_PER_W // 2
NVEC = N_OUT // L
QCHUNK = NVEC // 4
QWORDS = QCHUNK * L

DX = 1.0 / (N_IN - 1)
EPSILON = 1e-8
SCALE = 1.0 / (DX + EPSILON)
WQ = 65535.0
INV_WQ = 1.0 / WQ

_mesh = plsc.VectorSubcoreMesh(core_axis_name="c", subcore_axis_name="s")


@functools.partial(
    pl.kernel,
    out_type=jax.ShapeDtypeStruct((BATCH, N_OUT), jnp.float32),
    mesh=_mesh,
    compiler_params=pltpu.CompilerParams(needs_layout_passes=False),
    scratch_types=[
        pltpu.VMEM((N_IN,), jnp.float32),
        pltpu.VMEM((N_IN,), jnp.float32),
        pltpu.VMEM((N_IN,), jnp.float32),
        pltpu.VMEM((N_IN,), jnp.float32),
        pltpu.VMEM((2 * QWORDS,), jnp.float32),
        pltpu.VMEM((2 * QWORDS,), jnp.float32),
        pltpu.VMEM((2 * QWORDS,), jnp.float32),
        pltpu.VMEM((N_OUT,), jnp.int32),
        pltpu.VMEM((N_OUT,), jnp.int32),
        pltpu.SemaphoreType.DMA,
        pltpu.SemaphoreType.DMA,
        pltpu.SemaphoreType.DMA,
        pltpu.SemaphoreType.DMA,
        pltpu.SemaphoreType.DMA,
        pltpu.SemaphoreType.DMA,
        pltpu.SemaphoreType.DMA,
    ],
)
def _abacus_sc(act_hbm, sp_hbm, out_hbm,
               rb0, rb1, rb2, rb3, q0, q1, q2, tab0, tab1,
               si0, si1, si2, si3, so0, so1, so2):
    wid = lax.axis_index("c") * NS + lax.axis_index("s")
    iota2 = lax.iota(jnp.int32, L) * 2
    qbufs = (q0, q1, q2)
    souts = (so0, so1, so2)

    pltpu.sync_copy(sp_hbm.at[pl.ds(0, N_IN)], rb0)
    pltpu.sync_copy(sp_hbm.at[pl.ds(N_IN, N_IN)], rb1)

    def make_table(d, tab, src, jlo, jhi, qoff):
        @plsc.parallel_loop(jlo, jhi, unroll=4)
        def body(j):
            q = iota2 + (j * (2 * L) + d - qoff)
            v = plsc.load_gather(src, [q])
            v = jnp.clip(v, 0.0, 1.0)
            fi = (v * float(N_IN - 1)).astype(jnp.int32)
            fi = jnp.minimum(fi, N_IN - 2)
            xl = fi.astype(jnp.float32) * DX
            w = (v - xl) * SCALE
            w16 = (w * WQ + 0.5).astype(jnp.int32)
            tab[pl.ds(j * L, L)] = (fi << 16) | w16

    for d, tab in ((0, tab0), (1, tab1)):
        make_table(d, tab, rb0, 0, NVEC // 2, 0)
        make_table(d, tab, rb1, NVEC // 2, NVEC, N_IN)

    base = wid * ROWS_PER_W

    pltpu.async_copy(act_hbm.at[base], rb0, si0)
    pltpu.async_copy(act_hbm.at[base + 1], rb1, si1)

    def wait_slot(qb, sq):
        pltpu.make_async_copy(qb.at[pl.ds(0, QWORDS)],
                              out_hbm.at[base, pl.ds(0, QWORDS)], sq).wait()
        pltpu.make_async_copy(qb.at[pl.ds(0, QWORDS)],
                              out_hbm.at[base, pl.ds(0, QWORDS)], sq).wait()

    def do_pair(p, bu, bv, su, sv, pbu, pbv, psu, psv):
        ru = base + 2 * p
        rv = ru + 1
        pltpu.make_async_copy(act_hbm.at[ru], bu, su).wait()
        pltpu.make_async_copy(act_hbm.at[rv], bv, sv).wait()

        @pl.when(p < PAIRS_PER_W - 1)
        def _():
            pltpu.async_copy(act_hbm.at[ru + 2], pbu, psu)
            pltpu.async_copy(act_hbm.at[rv + 2], pbv, psv)

        for c in range(4):
            slot = c % 3
            qb = qbufs[slot]
            sq = souts[slot]
            if c < 3:
                @pl.when(p > 0)
                def _():
                    wait_slot(qb, sq)
            else:
                wait_slot(qb, sq)

            cbase = c * QWORDS

            @plsc.parallel_loop(c * QCHUNK, (c + 1) * QCHUNK, unroll=3)
            def inner(j):
                o = j * L
                lo = o - cbase
                p0 = tab0[pl.ds(o, L)]
                p1 = tab1[pl.ds(o, L)]
                i0 = p0 >> 16
                i1 = p1 >> 16
                i0r = i0 + 1
                i1r = i1 + 1
                a0 = (p0 & 0xFFFF).astype(jnp.float32) * INV_WQ
                a1 = (p1 & 0xFFFF).astype(jnp.float32) * INV_WQ
                u0l = plsc.load_gather(bu, [i0])
                u0r = plsc.load_gather(bu, [i0r])
                u1l = plsc.load_gather(bu, [i1])
                u1r = plsc.load_gather(bu, [i1r])
                v0l = plsc.load_gather(bv, [i0])
                v0r = plsc.load_gather(bv, [i0r])
                v1l = plsc.load_gather(bv, [i1])
                v1r = plsc.load_gather(bv, [i1r])
                tu0 = u0l + (u0r - u0l) * a0
                tu1 = u1l + (u1r - u1l) * a1
                tv0 = v0l + (v0r - v0l) * a0
                tv1 = v1l + (v1r - v1l) * a1
                qb[pl.ds(lo, L)] = (1.0 - tu0) * (1.0 - tu1)
                qb[pl.ds(QWORDS + lo, L)] = (1.0 - tv0) * (1.0 - tv1)

            pltpu.async_copy(qb.at[pl.ds(0, QWORDS)],
                             out_hbm.at[ru, pl.ds(cbase, QWORDS)], sq)
            pltpu.async_copy(qb.at[pl.ds(QWORDS, QWORDS)],
                             out_hbm.at[rv, pl.ds(cbase, QWORDS)], sq)

    def do_step(m, carry):
        do_pair(2 * m, rb0, rb1, si0, si1, rb2, rb3, si2, si3)
        do_pair(2 * m + 1, rb2, rb3, si2, si3, rb0, rb1, si0, si1)
        return carry

    lax.fori_loop(0, PAIRS_PER_W // 2, do_step, 0)

    for slot in range(3):
        wait_slot(qbufs[slot], souts[slot])


def kernel(activations, sample_points):
    sp_flat = sample_points.reshape(-1)
    return _abacus_sc(activations, sp_flat)


# speedup vs baseline: 1.0342x; 1.0342x over previous
import functools

import jax
import jax.numpy as jnp
from jax import lax
from jax.experimental import pallas as pl
from jax.experimental.pallas import tpu as pltpu
from jax.experimental.pallas import tpu_sc as plsc

N_IN = 16384
N_OUT = 16384
BATCH = 1024
DEGREE = 2

NC, NS, L = 2, 16, 16
NW = NC * NS
ROWS_PER_W = BATCH // NW
PAIRS_PER_W = ROWS_PER_W // 2
NVEC = N_OUT // L
QCHUNK = NVEC // 4
QWORDS = QCHUNK * L

DX = 1.0 / (N_IN - 1)
EPSILON = 1e-8
SCALE = 1.0 / (DX + EPSILON)
WQ = 65535.0
INV_WQ = 1.0 / WQ

_mesh = plsc.VectorSubcoreMesh(core_axis_name="c", subcore_axis_name="s")


@functools.partial(
    pl.kernel,
    out_type=jax.ShapeDtypeStruct((BATCH, N_OUT), jnp.float32),
    mesh=_mesh,
    compiler_params=pltpu.CompilerParams(needs_layout_passes=False),
    scratch_types=[
        pltpu.VMEM((N_IN,), jnp.float32),
        pltpu.VMEM((N_IN,), jnp.float32),
        pltpu.VMEM((N_IN,), jnp.float32),
        pltpu.VMEM((N_IN,), jnp.float32),
        pltpu.VMEM((2, QWORDS), jnp.float32),
        pltpu.VMEM((2, QWORDS), jnp.float32),
        pltpu.VMEM((2, QWORDS), jnp.float32),
        pltpu.VMEM((N_OUT,), jnp.int32),
        pltpu.VMEM((N_OUT,), jnp.int32),
        pltpu.SemaphoreType.DMA,
        pltpu.SemaphoreType.DMA,
        pltpu.SemaphoreType.DMA,
        pltpu.SemaphoreType.DMA,
        pltpu.SemaphoreType.DMA,
        pltpu.SemaphoreType.DMA,
        pltpu.SemaphoreType.DMA,
    ],
)
def _abacus_sc(act_hbm, sp_hbm, out_hbm,
               rb0, rb1, rb2, rb3, q0, q1, q2, tab0, tab1,
               si0, si1, si2, si3, so0, so1, so2):
    wid = lax.axis_index("c") * NS + lax.axis_index("s")
    iota2 = lax.iota(jnp.int32, L) * 2
    qbufs = (q0, q1, q2)
    souts = (so0, so1, so2)

    pltpu.async_copy(sp_hbm.at[pl.ds(0, N_IN)], rb0, si0)
    pltpu.async_copy(sp_hbm.at[pl.ds(N_IN, N_IN)], rb1, si1)
    pltpu.make_async_copy(sp_hbm.at[pl.ds(0, N_IN)], rb0, si0).wait()
    pltpu.make_async_copy(sp_hbm.at[pl.ds(N_IN, N_IN)], rb1, si1).wait()

    def pack_entry(v):
        v = jnp.clip(v, 0.0, 1.0)
        fi = (v * float(N_IN - 1)).astype(jnp.int32)
        fi = jnp.minimum(fi, N_IN - 2)
        xl = fi.astype(jnp.float32) * DX
        w = (v - xl) * SCALE
        w16 = (w * WQ + 0.5).astype(jnp.int32)
        return (fi << 16) | w16

    def make_tables(src, jlo, jhi, qoff):
        @plsc.parallel_loop(jlo, jhi, unroll=4)
        def body(j):
            q = iota2 + (j * (2 * L) - qoff)
            v0 = plsc.load_gather(src, [q])
            v1 = plsc.load_gather(src, [q + 1])
            tab0[pl.ds(j * L, L)] = pack_entry(v0)
            tab1[pl.ds(j * L, L)] = pack_entry(v1)

    make_tables(rb0, 0, NVEC // 2, 0)
    make_tables(rb1, NVEC // 2, NVEC, N_IN)

    base = wid * ROWS_PER_W

    pltpu.async_copy(act_hbm.at[base], rb0, si0)
    pltpu.async_copy(act_hbm.at[base + 1], rb1, si1)

    def wait_slot(qb, sq):
        pltpu.make_async_copy(
            qb, out_hbm.at[pl.ds(base, 2), pl.ds(0, QWORDS)], sq).wait()

    def do_pair(p, bu, bv, su, sv, pbu, pbv, psu, psv):
        ru = base + 2 * p
        rv = ru + 1
        pltpu.make_async_copy(act_hbm.at[ru], bu, su).wait()
        pltpu.make_async_copy(act_hbm.at[rv], bv, sv).wait()

        @pl.when(p < PAIRS_PER_W - 1)
        def _():
            pltpu.async_copy(act_hbm.at[ru + 2], pbu, psu)
            pltpu.async_copy(act_hbm.at[rv + 2], pbv, psv)

        for c in range(4):
            slot = c % 3
            qb = qbufs[slot]
            sq = souts[slot]
            if c < 3:
                @pl.when(p > 0)
                def _():
                    wait_slot(qb, sq)
            else:
                wait_slot(qb, sq)

            cbase = c * QWORDS

            @plsc.parallel_loop(c * QCHUNK, (c + 1) * QCHUNK, unroll=2)
            def inner(j):
                o = j * L
                lo = o - cbase
                p0 = tab0[pl.ds(o, L)]
                p1 = tab1[pl.ds(o, L)]
                i0 = p0 >> 16
                i1 = p1 >> 16
                i0r = i0 + 1
                i1r = i1 + 1
                a0 = (p0 & 0xFFFF).astype(jnp.float32) * INV_WQ
                a1 = (p1 & 0xFFFF).astype(jnp.float32) * INV_WQ
                u0l = plsc.load_gather(bu, [i0])
                u0r = plsc.load_gather(bu, [i0r])
                u1l = plsc.load_gather(bu, [i1])
                u1r = plsc.load_gather(bu, [i1r])
                v0l = plsc.load_gather(bv, [i0])
                v0r = plsc.load_gather(bv, [i0r])
                v1l = plsc.load_gather(bv, [i1])
                v1r = plsc.load_gather(bv, [i1r])
                tu0 = u0l + (u0r - u0l) * a0
                tu1 = u1l + (u1r - u1l) * a1
                tv0 = v0l + (v0r - v0l) * a0
                tv1 = v1l + (v1r - v1l) * a1
                qb[0, pl.ds(lo, L)] = (1.0 - tu0) * (1.0 - tu1)
                qb[1, pl.ds(lo, L)] = (1.0 - tv0) * (1.0 - tv1)

            pltpu.async_copy(
                qb, out_hbm.at[pl.ds(ru, 2), pl.ds(cbase, QWORDS)], sq)

    def do_step(m, carry):
        do_pair(2 * m, rb0, rb1, si0, si1, rb2, rb3, si2, si3)
        do_pair(2 * m + 1, rb2, rb3, si2, si3, rb0, rb1, si0, si1)
        return carry

    lax.fori_loop(0, PAIRS_PER_W // 2, do_step, 0)

    for slot in range(3):
        wait_slot(qbufs[slot], souts[slot])


def kernel(activations, sample_points):
    sp_flat = sample_points.reshape(-1)
    return _abacus_sc(activations, sp_flat)


# speedup vs baseline: 1.0348x; 1.0005x over previous
import functools

import jax
import jax.numpy as jnp
from jax import lax
from jax.experimental import pallas as pl
from jax.experimental.pallas import tpu as pltpu
from jax.experimental.pallas import tpu_sc as plsc

N_IN = 16384
N_OUT = 16384
BATCH = 1024
DEGREE = 2

NC, NS, L = 2, 16, 16
NW = NC * NS
ROWS_PER_W = BATCH // NW
PAIRS_PER_W = ROWS_PER_W // 2
NVEC = N_OUT // L
QCHUNK = NVEC // 4
QWORDS = QCHUNK * L

DX = 1.0 / (N_IN - 1)
EPSILON = 1e-8
SCALE = 1.0 / (DX + EPSILON)
WQ = 65535.0
INV_WQ = 1.0 / WQ

_mesh = plsc.VectorSubcoreMesh(core_axis_name="c", subcore_axis_name="s")


@functools.partial(
    pl.kernel,
    out_type=jax.ShapeDtypeStruct((BATCH, N_OUT), jnp.float32),
    mesh=_mesh,
    compiler_params=pltpu.CompilerParams(needs_layout_passes=False),
    scratch_types=[
        pltpu.VMEM((N_IN,), jnp.float32),
        pltpu.VMEM((N_IN,), jnp.float32),
        pltpu.VMEM((N_IN,), jnp.float32),
        pltpu.VMEM((N_IN,), jnp.float32),
        pltpu.VMEM((2, QWORDS), jnp.float32),
        pltpu.VMEM((2, QWORDS), jnp.float32),
        pltpu.VMEM((2, QWORDS), jnp.float32),
        pltpu.VMEM((N_OUT,), jnp.int32),
        pltpu.VMEM((N_OUT,), jnp.int32),
        pltpu.SemaphoreType.DMA,
        pltpu.SemaphoreType.DMA,
        pltpu.SemaphoreType.DMA,
        pltpu.SemaphoreType.DMA,
        pltpu.SemaphoreType.DMA,
        pltpu.SemaphoreType.DMA,
        pltpu.SemaphoreType.DMA,
    ],
)
def _abacus_sc(act_hbm, sp_hbm, out_hbm,
               rb0, rb1, rb2, rb3, q0, q1, q2, tab0, tab1,
               si0, si1, si2, si3, so0, so1, so2):
    wid = lax.axis_index("c") * NS + lax.axis_index("s")
    iota2 = lax.iota(jnp.int32, L) * 2
    qbufs = (q0, q1, q2)
    souts = (so0, so1, so2)

    pltpu.async_copy(sp_hbm.at[pl.ds(0, N_IN)], rb0, si0)
    pltpu.async_copy(sp_hbm.at[pl.ds(N_IN, N_IN)], rb1, si1)
    pltpu.make_async_copy(sp_hbm.at[pl.ds(0, N_IN)], rb0, si0).wait()
    pltpu.make_async_copy(sp_hbm.at[pl.ds(N_IN, N_IN)], rb1, si1).wait()

    def pack_entry(v):
        v = jnp.clip(v, 0.0, 1.0)
        fi = (v * float(N_IN - 1)).astype(jnp.int32)
        fi = jnp.minimum(fi, N_IN - 2)
        xl = fi.astype(jnp.float32) * DX
        w = (v - xl) * SCALE
        w16 = (w * WQ + 0.5).astype(jnp.int32)
        return (fi << 16) | w16

    def make_tables(src, jlo, jhi, qoff):
        @plsc.parallel_loop(jlo, jhi, unroll=8)
        def body(j):
            q = iota2 + (j * (2 * L) - qoff)
            v0 = plsc.load_gather(src, [q])
            v1 = plsc.load_gather(src, [q + 1])
            tab0[pl.ds(j * L, L)] = pack_entry(v0)
            tab1[pl.ds(j * L, L)] = pack_entry(v1)

    make_tables(rb0, 0, NVEC // 2, 0)
    make_tables(rb1, NVEC // 2, NVEC, N_IN)

    base = wid * ROWS_PER_W

    pltpu.async_copy(act_hbm.at[base], rb0, si0)
    pltpu.async_copy(act_hbm.at[base + 1], rb1, si1)

    def wait_slot(qb, sq):
        pltpu.make_async_copy(
            qb, out_hbm.at[pl.ds(base, 2), pl.ds(0, QWORDS)], sq).wait()

    def do_pair(p, bu, bv, su, sv, pbu, pbv, psu, psv):
        ru = base + 2 * p
        rv = ru + 1
        pltpu.make_async_copy(act_hbm.at[ru], bu, su).wait()
        pltpu.make_async_copy(act_hbm.at[rv], bv, sv).wait()

        @pl.when(p < PAIRS_PER_W - 1)
        def _():
            pltpu.async_copy(act_hbm.at[ru + 2], pbu, psu)
            pltpu.async_copy(act_hbm.at[rv + 2], pbv, psv)

        for c in range(4):
            slot = c % 3
            qb = qbufs[slot]
            sq = souts[slot]
            if c < 3:
                @pl.when(p > 0)
                def _():
                    wait_slot(qb, sq)
            else:
                wait_slot(qb, sq)

            cbase = c * QWORDS

            @plsc.parallel_loop(c * QCHUNK, (c + 1) * QCHUNK, unroll=2)
            def inner(j):
                o = j * L
                lo = o - cbase
                p0 = tab0[pl.ds(o, L)]
                p1 = tab1[pl.ds(o, L)]
                i0 = p0 >> 16
                i1 = p1 >> 16
                i0r = i0 + 1
                i1r = i1 + 1
                a0 = (p0 & 0xFFFF).astype(jnp.float32) * INV_WQ
                a1 = (p1 & 0xFFFF).astype(jnp.float32) * INV_WQ
                u0l = plsc.load_gather(bu, [i0])
                u0r = plsc.load_gather(bu, [i0r])
                u1l = plsc.load_gather(bu, [i1])
                u1r = plsc.load_gather(bu, [i1r])
                v0l = plsc.load_gather(bv, [i0])
                v0r = plsc.load_gather(bv, [i0r])
                v1l = plsc.load_gather(bv, [i1])
                v1r = plsc.load_gather(bv, [i1r])
                tu0 = u0l + (u0r - u0l) * a0
                tu1 = u1l + (u1r - u1l) * a1
                tv0 = v0l + (v0r - v0l) * a0
                tv1 = v1l + (v1r - v1l) * a1
                qb[0, pl.ds(lo, L)] = (1.0 - tu0) * (1.0 - tu1)
                qb[1, pl.ds(lo, L)] = (1.0 - tv0) * (1.0 - tv1)

            pltpu.async_copy(
                qb, out_hbm.at[pl.ds(ru, 2), pl.ds(cbase, QWORDS)], sq)

    def do_step(m, carry):
        do_pair(2 * m, rb0, rb1, si0, si1, rb2, rb3, si2, si3)
        do_pair(2 * m + 1, rb2, rb3, si2, si3, rb0, rb1, si0, si1)
        return carry

    lax.fori_loop(0, PAIRS_PER_W // 2, do_step, 0)

    for slot in range(3):
        wait_slot(qbufs[slot], souts[slot])


def kernel(activations, sample_points):
    sp_flat = sample_points.reshape(-1)
    return _abacus_sc(activations, sp_flat)


# speedup vs baseline: 1.0498x; 1.0145x over previous
import functools

import jax
import jax.numpy as jnp
from jax import lax
from jax.experimental import pallas as pl
from jax.experimental.pallas import tpu as pltpu
from jax.experimental.pallas import tpu_sc as plsc

N_IN = 16384
N_OUT = 16384
BATCH = 1024
DEGREE = 2

NC, NS, L = 2, 16, 16
NW = NC * NS
ROWS_PER_W = BATCH // NW
PAIRS_PER_W = ROWS_PER_W // 2
NVEC = N_OUT // L
QCHUNK = NVEC // 4
QWORDS = QCHUNK * L

DX = 1.0 / (N_IN - 1)
EPSILON = 1e-8
SCALE = 1.0 / (DX + EPSILON)
WQ = 65535.0
INV_WQ = 1.0 / WQ

_mesh = plsc.VectorSubcoreMesh(core_axis_name="c", subcore_axis_name="s")


@functools.partial(
    pl.kernel,
    out_type=jax.ShapeDtypeStruct((BATCH, N_OUT), jnp.float32),
    mesh=_mesh,
    compiler_params=pltpu.CompilerParams(needs_layout_passes=False),
    scratch_types=[
        pltpu.VMEM((N_IN,), jnp.float32),
        pltpu.VMEM((N_IN,), jnp.float32),
        pltpu.VMEM((N_IN,), jnp.float32),
        pltpu.VMEM((N_IN,), jnp.float32),
        pltpu.VMEM((2, QWORDS), jnp.float32),
        pltpu.VMEM((2, QWORDS), jnp.float32),
        pltpu.VMEM((2, QWORDS), jnp.float32),
        pltpu.VMEM((N_OUT,), jnp.int32),
        pltpu.VMEM((N_OUT,), jnp.int32),
        pltpu.SemaphoreType.DMA,
        pltpu.SemaphoreType.DMA,
        pltpu.SemaphoreType.DMA,
        pltpu.SemaphoreType.DMA,
        pltpu.SemaphoreType.DMA,
        pltpu.SemaphoreType.DMA,
        pltpu.SemaphoreType.DMA,
    ],
)
def _abacus_sc(act_hbm, sp_hbm, out_hbm,
               rb0, rb1, rb2, rb3, q0, q1, q2, tab0, tab1,
               si0, si1, si2, si3, so0, so1, so2):
    wid = lax.axis_index("c") * NS + lax.axis_index("s")
    iota2 = lax.iota(jnp.int32, L) * 2
    qbufs = (q0, q1, q2)
    souts = (so0, so1, so2)

    pltpu.async_copy(sp_hbm.at[pl.ds(0, N_IN)], rb0, si0)
    pltpu.async_copy(sp_hbm.at[pl.ds(N_IN, N_IN)], rb1, si1)
    pltpu.make_async_copy(sp_hbm.at[pl.ds(0, N_IN)], rb0, si0).wait()
    pltpu.make_async_copy(sp_hbm.at[pl.ds(N_IN, N_IN)], rb1, si1).wait()

    def pack_entry(v):
        v = jnp.clip(v, 0.0, 1.0)
        fi = (v * float(N_IN - 1)).astype(jnp.int32)
        fi = jnp.minimum(fi, N_IN - 2)
        xl = fi.astype(jnp.float32) * DX
        w = (v - xl) * SCALE
        wb = lax.bitcast_convert_type(w, jnp.int32)
        wb = (wb + 0x8000) & jnp.int32(-65536)
        return wb | fi

    def make_tables(src, jlo, jhi, qoff):
        @plsc.parallel_loop(jlo, jhi, unroll=8)
        def body(j):
            q = iota2 + (j * (2 * L) - qoff)
            v0 = plsc.load_gather(src, [q])
            v1 = plsc.load_gather(src, [q + 1])
            tab0[pl.ds(j * L, L)] = pack_entry(v0)
            tab1[pl.ds(j * L, L)] = pack_entry(v1)

    make_tables(rb0, 0, NVEC // 2, 0)
    make_tables(rb1, NVEC // 2, NVEC, N_IN)

    base = wid * ROWS_PER_W

    pltpu.async_copy(act_hbm.at[base], rb0, si0)
    pltpu.async_copy(act_hbm.at[base + 1], rb1, si1)

    def wait_slot(qb, sq):
        pltpu.make_async_copy(
            qb, out_hbm.at[pl.ds(base, 2), pl.ds(0, QWORDS)], sq).wait()

    def do_pair(p, bu, bv, su, sv, pbu, pbv, psu, psv):
        ru = base + 2 * p
        rv = ru + 1
        pltpu.make_async_copy(act_hbm.at[ru], bu, su).wait()
        pltpu.make_async_copy(act_hbm.at[rv], bv, sv).wait()

        @pl.when(p < PAIRS_PER_W - 1)
        def _():
            pltpu.async_copy(act_hbm.at[ru + 2], pbu, psu)
            pltpu.async_copy(act_hbm.at[rv + 2], pbv, psv)

        for c in range(4):
            slot = c % 3
            qb = qbufs[slot]
            sq = souts[slot]
            if c < 3:
                @pl.when(p > 0)
                def _():
                    wait_slot(qb, sq)
            else:
                wait_slot(qb, sq)

            cbase = c * QWORDS

            @plsc.parallel_loop(c * QCHUNK, (c + 1) * QCHUNK, unroll=2)
            def inner(j):
                o = j * L
                lo = o - cbase
                p0 = tab0[pl.ds(o, L)]
                p1 = tab1[pl.ds(o, L)]
                i0 = p0 & 0xFFFF
                i1 = p1 & 0xFFFF
                i0r = i0 + 1
                i1r = i1 + 1
                a0 = lax.bitcast_convert_type(p0 & jnp.int32(-65536),
                                              jnp.float32)
                a1 = lax.bitcast_convert_type(p1 & jnp.int32(-65536),
                                              jnp.float32)
                u0l = plsc.load_gather(bu, [i0])
                u0r = plsc.load_gather(bu, [i0r])
                u1l = plsc.load_gather(bu, [i1])
                u1r = plsc.load_gather(bu, [i1r])
                v0l = plsc.load_gather(bv, [i0])
                v0r = plsc.load_gather(bv, [i0r])
                v1l = plsc.load_gather(bv, [i1])
                v1r = plsc.load_gather(bv, [i1r])
                tu0 = u0l + (u0r - u0l) * a0
                tu1 = u1l + (u1r - u1l) * a1
                tv0 = v0l + (v0r - v0l) * a0
                tv1 = v1l + (v1r - v1l) * a1
                qb[0, pl.ds(lo, L)] = (1.0 - tu0) * (1.0 - tu1)
                qb[1, pl.ds(lo, L)] = (1.0 - tv0) * (1.0 - tv1)

            pltpu.async_copy(
                qb, out_hbm.at[pl.ds(ru, 2), pl.ds(cbase, QWORDS)], sq)

    def do_step(m, carry):
        do_pair(2 * m, rb0, rb1, si0, si1, rb2, rb3, si2, si3)
        do_pair(2 * m + 1, rb2, rb3, si2, si3, rb0, rb1, si0, si1)
        return carry

    lax.fori_loop(0, PAIRS_PER_W // 2, do_step, 0)

    for slot in range(3):
        wait_slot(qbufs[slot], souts[slot])


def kernel(activations, sample_points):
    sp_flat = sample_points.reshape(-1)
    return _abacus_sc(activations, sp_flat)
